# Initial kernel scaffold; baseline (speedup 1.0000x reference)
import jax
import jax.numpy as jnp
from jax.experimental import pallas as pl


def kernel(hashes, weights, table):
    raise NotImplementedError("write your pallas kernel here")


# speedup vs baseline: 2.6286x; 2.6286x over previous
import functools

import jax
import jax.numpy as jnp
from jax import lax
from jax.experimental import pallas as pl
from jax.experimental.pallas import tpu as pltpu
from jax.experimental.pallas import tpu_sc as plsc

NUM_EMBEDDINGS = 1000000
D = 32
PADDING_IDX = 0
B = 16384
HIST = 50

L = 16
NC, NS = 2, 16
NW = NC * NS
RW = B // NW
CB = 32
GC = CB * HIST
NCHUNK = RW // CB


def _body(hashes_hbm, wts_hbm, table_hbm, out_hbm, idx_v, wts_v, rows_v,
          outb_v, sem):
    wid = lax.axis_index("s") * NC + lax.axis_index("c")

    def chunk_body(g, carry):
        base_b = wid * RW + g * CB
        base_g = base_b * HIST
        pltpu.sync_copy(hashes_hbm.at[pl.ds(base_g, GC)], idx_v)
        pltpu.sync_copy(wts_hbm.at[pl.ds(base_g, GC)], wts_v.at[pl.ds(0, GC)])

        def wm_body(j, c):
            iv = idx_v[pl.ds(j * L, L)]
            wv = wts_v[pl.ds(j * L, L)]
            wts_v[pl.ds(j * L, L)] = jnp.where(iv == PADDING_IDX, 0.0, wv)
            return c

        lax.fori_loop(0, GC // L, wm_body, 0)

        pltpu.async_copy(table_hbm.at[idx_v], rows_v, sem).wait()

        def row_body(b, c):
            r0 = b * HIST

            def k_body(k, acc):
                a0, a1 = acc
                rk = r0 + k * L
                w16 = wts_v[pl.ds(rk, L)]
                for j in range(L):
                    w = w16[j]
                    a0 = a0 + w * rows_v[rk + j, pl.ds(0, L)]
                    a1 = a1 + w * rows_v[rk + j, pl.ds(L, L)]
                return (a0, a1)

            zero = jnp.zeros((L,), jnp.float32)
            a0, a1 = lax.fori_loop(0, HIST // L, k_body, (zero, zero))
            rt = r0 + (HIST // L) * L
            wt = wts_v[pl.ds(rt, L)]
            for j in range(HIST % L):
                w = wt[j]
                a0 = a0 + w * rows_v[rt + j, pl.ds(0, L)]
                a1 = a1 + w * rows_v[rt + j, pl.ds(L, L)]
            outb_v[b, pl.ds(0, L)] = a0
            outb_v[b, pl.ds(L, L)] = a1
            return c

        lax.fori_loop(0, CB, row_body, 0)
        pltpu.sync_copy(outb_v, out_hbm.at[pl.ds(base_b, CB)])
        return carry

    lax.fori_loop(0, NCHUNK, chunk_body, 0)


@jax.jit
def kernel(hashes, weights, table):
    hashes_flat = hashes.astype(jnp.int32).reshape(B * HIST)
    weights_flat = weights.reshape(B * HIST)
    mesh = plsc.VectorSubcoreMesh(core_axis_name="c", subcore_axis_name="s")
    run = pl.kernel(
        _body,
        out_type=jax.ShapeDtypeStruct((B, D), jnp.float32),
        mesh=mesh,
        scratch_types=[
            pltpu.VMEM((GC,), jnp.int32),
            pltpu.VMEM((GC + L,), jnp.float32),
            pltpu.VMEM((GC, D), jnp.float32),
            pltpu.VMEM((CB, D), jnp.float32),
            pltpu.SemaphoreType.DMA,
        ],
        compiler_params=pltpu.CompilerParams(use_tc_tiling_on_sc=False),
    )
    return run(hashes_flat, weights_flat, table)


# speedup vs baseline: 2.8446x; 1.0822x over previous
import jax
import jax.numpy as jnp
from jax import lax
from jax.experimental import pallas as pl
from jax.experimental.pallas import tpu as pltpu
from jax.experimental.pallas import tpu_sc as plsc

NUM_EMBEDDINGS = 1000000
D = 32
PADDING_IDX = 0
B = 16384
HIST = 50

L = 16
NC, NS = 2, 16
NW = NC * NS
RW = B // NW
CB = 32
GC = CB * HIST
NCHUNK = RW // CB


def _body(hashes_hbm, wts_hbm, table_hbm, out_hbm,
          idx0, wts0, rows0, idx1, wts1, rows1, outb_v, sem0, sem1):
    wid = lax.axis_index("s") * NC + lax.axis_index("c")
    idx = (idx0, idx1)
    wts = (wts0, wts1)
    rows = (rows0, rows1)
    sem = (sem0, sem1)

    def stage(g, p):
        base_g = (wid * RW + g * CB) * HIST
        pltpu.sync_copy(hashes_hbm.at[pl.ds(base_g, GC)], idx[p])
        pltpu.sync_copy(wts_hbm.at[pl.ds(base_g, GC)],
                        wts[p].at[pl.ds(0, GC)])

        def wm_body(j, c):
            iv = idx[p][pl.ds(j * L, L)]
            wv = wts[p][pl.ds(j * L, L)]
            wts[p][pl.ds(j * L, L)] = jnp.where(iv == PADDING_IDX, 0.0, wv)
            return c

        lax.fori_loop(0, GC // L, wm_body, 0)
        pltpu.async_copy(table_hbm.at[idx[p]], rows[p], sem[p])

    def consume(g, p):
        pltpu.make_async_copy(table_hbm.at[idx[p]], rows[p], sem[p]).wait()
        rv, wv = rows[p], wts[p]

        def row_body(b, c):
            r0 = b * HIST

            def k_body(k, acc):
                a0, a1, b0, b1 = acc
                rk = r0 + k * L
                w16 = wv[pl.ds(rk, L)]
                for j in range(0, L, 2):
                    w = w16[j]
                    a0 = a0 + w * rv[rk + j, pl.ds(0, L)]
                    a1 = a1 + w * rv[rk + j, pl.ds(L, L)]
                    w2 = w16[j + 1]
                    b0 = b0 + w2 * rv[rk + j + 1, pl.ds(0, L)]
                    b1 = b1 + w2 * rv[rk + j + 1, pl.ds(L, L)]
                return (a0, a1, b0, b1)

            z = jnp.zeros((L,), jnp.float32)
            a0, a1, b0, b1 = lax.fori_loop(0, HIST // L, k_body, (z, z, z, z))
            rt = r0 + (HIST // L) * L
            wt16 = wv[pl.ds(rt, L)]
            a0 = a0 + wt16[0] * rv[rt, pl.ds(0, L)]
            a1 = a1 + wt16[0] * rv[rt, pl.ds(L, L)]
            b0 = b0 + wt16[1] * rv[rt + 1, pl.ds(0, L)]
            b1 = b1 + wt16[1] * rv[rt + 1, pl.ds(L, L)]
            outb_v[b, pl.ds(0, L)] = a0 + b0
            outb_v[b, pl.ds(L, L)] = a1 + b1
            return c

        lax.fori_loop(0, CB, row_body, 0)
        base_b = wid * RW + g * CB
        pltpu.sync_copy(outb_v, out_hbm.at[pl.ds(base_b, CB)])

    stage(0, 0)

    def outer(gb, c):
        for p in range(2):
            g = 2 * gb + p

            @pl.when(g + 1 < NCHUNK)
            def _():
                stage(g + 1, 1 - p)

            consume(g, p)
        return c

    lax.fori_loop(0, NCHUNK // 2, outer, 0)


@jax.jit
def kernel(hashes, weights, table):
    hashes_flat = hashes.astype(jnp.int32).reshape(B * HIST)
    weights_flat = weights.reshape(B * HIST)
    mesh = plsc.VectorSubcoreMesh(core_axis_name="c", subcore_axis_name="s")
    run = pl.kernel(
        _body,
        out_type=jax.ShapeDtypeStruct((B, D), jnp.float32),
        mesh=mesh,
        scratch_types=[
            pltpu.VMEM((GC,), jnp.int32),
            pltpu.VMEM((GC + L,), jnp.float32),
            pltpu.VMEM((GC, D), jnp.float32),
            pltpu.VMEM((GC,), jnp.int32),
            pltpu.VMEM((GC + L,), jnp.float32),
            pltpu.VMEM((GC, D), jnp.float32),
            pltpu.VMEM((CB, D), jnp.float32),
            pltpu.SemaphoreType.DMA,
            pltpu.SemaphoreType.DMA,
        ],
        compiler_params=pltpu.CompilerParams(use_tc_tiling_on_sc=False),
    )
    return run(hashes_flat, weights_flat, table)
